# baseline (speedup 1.0000x reference)
import jax
import jax.numpy as jnp
from jax.experimental import pallas as pl
from jax.experimental.pallas import tpu as pltpu

_B, _T, _D = 16384, 100, 64
_LANES = 128
_MROWS = (_B * _T) // _LANES


def _body(x_hbm, y_hbm, mask_ref):
    mask_ref[...] = jnp.zeros_like(mask_ref)


def kernel(inputs):
    y, mask2d = pl.pallas_call(
        _body,
        out_shape=(
            jax.ShapeDtypeStruct((_B, _T, _D), inputs.dtype),
            jax.ShapeDtypeStruct((_MROWS, _LANES), inputs.dtype),
        ),
        in_specs=[pl.BlockSpec(memory_space=pltpu.MemorySpace.HBM)],
        out_specs=(
            pl.BlockSpec(memory_space=pltpu.MemorySpace.HBM),
            pl.BlockSpec((_MROWS, _LANES), lambda: (0, 0)),
        ),
        input_output_aliases={0: 0},
    )(inputs)
    return (y, mask2d.reshape(_B, _T, 1))
